# baseline (speedup 1.0000x reference)
import functools

import jax
import jax.numpy as jnp
from jax import lax
from jax.experimental import pallas as pl
from jax.experimental.pallas import tpu as pltpu
from jax.experimental.pallas import tpu_sc as plsc

_D = 64
_C = 128


@functools.cache
def _make_gather(B):
    info = plsc.get_sparse_core_info()
    nc, ns = info.num_cores, info.num_subcores
    nw = nc * ns
    b_per_w = B // nw
    n_chunks = b_per_w // _C
    mesh = plsc.VectorSubcoreMesh(core_axis_name="c", subcore_axis_name="s")

    @functools.partial(
        pl.kernel,
        out_type=jax.ShapeDtypeStruct((B, _D), jnp.float32),
        mesh=mesh,
        scratch_types=[
            pltpu.VMEM((n_chunks, _C), jnp.int32),
            pltpu.VMEM((_C, _D), jnp.float32),
            pltpu.SemaphoreType.DMA,
        ],
        compiler_params=pltpu.CompilerParams(use_tc_tiling_on_sc=False),
    )
    def gather_k(idx_hbm, table_hbm, out_hbm, idx_v, buf, sem):
        wid = lax.axis_index("s") * nc + lax.axis_index("c")
        cb = wid * n_chunks
        pltpu.sync_copy(idx_hbm.at[pl.ds(cb, n_chunks)], idx_v)

        def body(g, carry):
            pltpu.async_copy(table_hbm.at[idx_v.at[g]], buf, sem).wait()
            pltpu.sync_copy(buf, out_hbm.at[pl.ds((cb + g) * _C, _C)])
            return carry

        lax.fori_loop(0, n_chunks, body, 0)

    return gather_k


def kernel(word_inputs, table):
    batch, seq = word_inputs.shape
    b = batch * seq
    idx2d = word_inputs.reshape(b // _C, _C).astype(jnp.int32)
    out = _make_gather(b)(idx2d, table)
    return out.reshape(batch, seq, _D)


# speedup vs baseline: 1.1137x; 1.1137x over previous
import functools

import jax
import jax.numpy as jnp
from jax import lax
from jax.experimental import pallas as pl
from jax.experimental.pallas import tpu as pltpu
from jax.experimental.pallas import tpu_sc as plsc

_D = 64
_C = 128
_NBUF = 4


@functools.cache
def _make_gather(B):
    info = plsc.get_sparse_core_info()
    nc, ns = info.num_cores, info.num_subcores
    nw = nc * ns
    b_per_w = B // nw
    n_chunks = b_per_w // _C
    assert n_chunks % _NBUF == 0
    mesh = plsc.VectorSubcoreMesh(core_axis_name="c", subcore_axis_name="s")

    @functools.partial(
        pl.kernel,
        out_type=jax.ShapeDtypeStruct((B, _D), jnp.float32),
        mesh=mesh,
        scratch_types=[
            pltpu.VMEM((n_chunks, _C), jnp.int32),
            pltpu.VMEM((_NBUF, _C, _D), jnp.float32),
            pltpu.SemaphoreType.DMA((_NBUF,)),
            pltpu.SemaphoreType.DMA((_NBUF,)),
        ],
        compiler_params=pltpu.CompilerParams(use_tc_tiling_on_sc=False),
    )
    def gather_k(idx_hbm, table_hbm, out_hbm, idx_v, bufs, gsem, wsem):
        wid = lax.axis_index("s") * nc + lax.axis_index("c")
        cb = wid * n_chunks
        pltpu.sync_copy(idx_hbm.at[pl.ds(cb, n_chunks)], idx_v)

        def fire_gather(g, b):
            pltpu.async_copy(table_hbm.at[idx_v.at[g]], bufs.at[b], gsem.at[b])

        def wait_gather(g, b):
            pltpu.make_async_copy(
                table_hbm.at[idx_v.at[g]], bufs.at[b], gsem.at[b]).wait()

        def fire_wb(g, b):
            pltpu.async_copy(
                bufs.at[b], out_hbm.at[pl.ds((cb + g) * _C, _C)], wsem.at[b])

        def wait_wb(g, b):
            pltpu.make_async_copy(
                bufs.at[b], out_hbm.at[pl.ds((cb + g) * _C, _C)],
                wsem.at[b]).wait()

        for b in range(_NBUF):
            fire_gather(b, b)

        @pl.loop(0, n_chunks - _NBUF, step=_NBUF)
        def _round(gg):
            for b in range(_NBUF):
                wait_gather(gg + b, b)
                fire_wb(gg + b, b)
            for b in range(_NBUF):
                wait_wb(gg + b, b)
                fire_gather(gg + _NBUF + b, b)

        last = n_chunks - _NBUF
        for b in range(_NBUF):
            wait_gather(last + b, b)
            fire_wb(last + b, b)
        for b in range(_NBUF):
            wait_wb(last + b, b)

    return gather_k


def kernel(word_inputs, table):
    batch, seq = word_inputs.shape
    b = batch * seq
    idx2d = word_inputs.reshape(b // _C, _C).astype(jnp.int32)
    out = _make_gather(b)(idx2d, table)
    return out.reshape(batch, seq, _D)


# speedup vs baseline: 1.1162x; 1.0022x over previous
import functools

import jax
import jax.numpy as jnp
from jax import lax
from jax.experimental import pallas as pl
from jax.experimental.pallas import tpu as pltpu
from jax.experimental.pallas import tpu_sc as plsc

_D = 64
_NBUF = 4


@functools.cache
def _make_gather(batch, seq):
    info = plsc.get_sparse_core_info()
    nc, ns = info.num_cores, info.num_subcores
    nw = nc * ns
    rows_per_w = batch // nw
    n_sub = 5
    sub = seq // n_sub
    mesh = plsc.VectorSubcoreMesh(core_axis_name="c", subcore_axis_name="s")

    @functools.partial(
        pl.kernel,
        out_type=jax.ShapeDtypeStruct((batch, seq, _D), jnp.float32),
        mesh=mesh,
        scratch_types=[
            pltpu.VMEM((rows_per_w, seq), jnp.int32),
            pltpu.VMEM((_NBUF, seq, _D), jnp.float32),
            pltpu.SemaphoreType.DMA((_NBUF,)),
            pltpu.SemaphoreType.DMA((_NBUF,)),
        ],
        compiler_params=pltpu.CompilerParams(use_tc_tiling_on_sc=False),
    )
    def gather_k(idx_hbm, table_hbm, out_hbm, idx_v, bufs, gsem, wsem):
        wid = lax.axis_index("s") * nc + lax.axis_index("c")
        row0 = wid * rows_per_w
        pltpu.sync_copy(idx_hbm.at[pl.ds(row0, rows_per_w)], idx_v)

        def fire_gathers(r, b):
            for h in range(n_sub):
                pltpu.async_copy(
                    table_hbm.at[idx_v.at[r, pl.ds(h * sub, sub)]],
                    bufs.at[b, pl.ds(h * sub, sub)], gsem.at[b])

        def wait_gathers(r, b):
            for h in range(n_sub):
                pltpu.make_async_copy(
                    table_hbm.at[idx_v.at[r, pl.ds(h * sub, sub)]],
                    bufs.at[b, pl.ds(h * sub, sub)], gsem.at[b]).wait()

        def fire_wb(r, b):
            pltpu.async_copy(bufs.at[b], out_hbm.at[row0 + r], wsem.at[b])

        def wait_wb(r, b):
            pltpu.make_async_copy(
                bufs.at[b], out_hbm.at[row0 + r], wsem.at[b]).wait()

        for b in range(_NBUF):
            fire_gathers(b, b)

        @pl.loop(0, rows_per_w - _NBUF, step=_NBUF)
        def _round(rr):
            for b in range(_NBUF):
                wait_gathers(rr + b, b)
                fire_wb(rr + b, b)
            for b in range(_NBUF):
                wait_wb(rr + b, b)
                fire_gathers(rr + _NBUF + b, b)

        last = rows_per_w - _NBUF
        for b in range(_NBUF):
            wait_gathers(last + b, b)
            fire_wb(last + b, b)
        for b in range(_NBUF):
            wait_wb(last + b, b)

    return gather_k


def kernel(word_inputs, table):
    batch, seq = word_inputs.shape
    return _make_gather(batch, seq)(word_inputs.astype(jnp.int32), table)
